# baseline (speedup 1.0000x reference)
import functools

import jax
import jax.numpy as jnp
from jax import lax
from jax.experimental import pallas as pl
from jax.experimental.pallas import tpu as pltpu
from jax.experimental.pallas import tpu_sc as plsc

_B = 4
_S = 8192
_D = 2048
_N = 64
_STRIDE = 128

_mesh = plsc.ScalarSubcoreMesh(axis_name="c", num_cores=2)


@functools.partial(
    pl.kernel,
    mesh=_mesh,
    out_type=jax.ShapeDtypeStruct((_B, _N, _D), jnp.float32),
)
def _gather_scs(x_hbm, out_hbm):
    c = lax.axis_index("c")
    for j in range(_B // 2):
        b = c * (_B // 2) + j
        pltpu.sync_copy(x_hbm.at[b, :, 0, :], out_hbm.at[b])


def kernel(x):
    return _gather_scs(x.reshape(_B, _N, _STRIDE, _D))


# speedup vs baseline: 13.6442x; 13.6442x over previous
import jax
import jax.numpy as jnp
from jax.experimental import pallas as pl
from jax.experimental.pallas import tpu as pltpu

_B = 4
_S = 8192
_D = 2048
_N = 64
_STRIDE = 128


def _gather_body(x_hbm, o_ref, sem):
    b = pl.program_id(0)
    pltpu.make_async_copy(x_hbm.at[b, :, 0, :], o_ref.at[0], sem).start()
    pltpu.make_async_copy(x_hbm.at[b, :, 0, :], o_ref.at[0], sem).wait()


def kernel(x):
    x4 = x.reshape(_B, _N, _STRIDE, _D)
    out = pl.pallas_call(
        _gather_body,
        grid=(_B,),
        in_specs=[pl.BlockSpec(memory_space=pl.ANY)],
        out_specs=pl.BlockSpec((1, _N, _D), lambda b: (b, 0, 0)),
        out_shape=jax.ShapeDtypeStruct((_B, _N, _D), jnp.float32),
        scratch_shapes=[pltpu.SemaphoreType.DMA],
    )(x4)
    return out


# speedup vs baseline: 30.5619x; 2.2399x over previous
import jax
import jax.numpy as jnp
from jax.experimental import pallas as pl
from jax.experimental.pallas import tpu as pltpu

_B = 4
_S = 8192
_D = 2048
_N = 64
_STRIDE = 128
_ROWS = _B * _N
_C = 8
_RPC = _ROWS // _C
_HPB = _N // _RPC


def _read(x_hbm, buf, rsem, c):
    b, h = divmod(c, _HPB)
    return pltpu.make_async_copy(
        x_hbm.at[b, pl.ds(h * _RPC, _RPC), 0, :],
        buf.at[pl.ds(c * _RPC, _RPC)],
        rsem.at[c],
    )


def _write(buf, out_hbm, wsem, c):
    return pltpu.make_async_copy(
        buf.at[pl.ds(c * _RPC, _RPC)],
        out_hbm.at[pl.ds(c * _RPC, _RPC)],
        wsem.at[c],
    )


def _gather_body(x_hbm, out_hbm, buf, rsem, wsem):
    for c in range(_C):
        _read(x_hbm, buf, rsem, c).start()
    for c in range(_C):
        _read(x_hbm, buf, rsem, c).wait()
        _write(buf, out_hbm, wsem, c).start()
    for c in range(_C):
        _write(buf, out_hbm, wsem, c).wait()


def kernel(x):
    x4 = x.reshape(_B, _N, _STRIDE, _D)
    out = pl.pallas_call(
        _gather_body,
        in_specs=[pl.BlockSpec(memory_space=pl.ANY)],
        out_specs=pl.BlockSpec(memory_space=pl.ANY),
        out_shape=jax.ShapeDtypeStruct((_ROWS, _D), jnp.float32),
        scratch_shapes=[
            pltpu.VMEM((_ROWS, _D), jnp.float32),
            pltpu.SemaphoreType.DMA((_C,)),
            pltpu.SemaphoreType.DMA((_C,)),
        ],
    )(x4)
    return out.reshape(_B, _N, _D)
